# baseline (speedup 1.0000x reference)
import jax
import jax.numpy as jnp
from jax.experimental import pallas as pl
from jax.experimental.pallas import tpu as pltpu

P = 10000
U = 10000
D = 128
BM = 400


def _mm_body(a_ref, b_ref, o_ref):
    o_ref[...] = jax.lax.dot_general(
        a_ref[...], b_ref[...],
        dimension_numbers=(((1,), (0,)), ((), ())),
        preferred_element_type=jnp.float32,
    )


def _mm(a, b):
    m, k = a.shape
    _, n = b.shape
    return pl.pallas_call(
        _mm_body,
        grid=(m // BM,),
        in_specs=[
            pl.BlockSpec((BM, k), lambda i: (i, 0)),
            pl.BlockSpec((k, n), lambda i: (0, 0)),
        ],
        out_specs=pl.BlockSpec((BM, n), lambda i: (i, 0)),
        out_shape=jax.ShapeDtypeStruct((m, n), jnp.float32),
        compiler_params=pltpu.CompilerParams(
            dimension_semantics=("parallel",),
        ),
    )(a, b)


@jax.jit
def _fused(pois_embs, HG_up, HG_pu):
    tmp = _mm(HG_up, pois_embs)
    return _mm(HG_pu, tmp)


def kernel(pois_embs, pad_all_train_sessions, HG_up, HG_pu):
    del pad_all_train_sessions
    return _fused(pois_embs, HG_up, HG_pu)
